# baseline (speedup 1.0000x reference)
import functools

import jax
import jax.numpy as jnp
from jax import lax
from jax.experimental import pallas as pl
from jax.experimental.pallas import tpu as pltpu
from jax.experimental.pallas import tpu_sc as plsc

POOL_SIZE = 65536
N_TOK = 16384
NOPE_DIM = 512
ROPE_DIM = 64
ROW_DIM = NOPE_DIM + ROPE_DIM

_NC, _NS = 2, 16
_NW = _NC * _NS
_NBAND = _NW // 2
_COLS_B = POOL_SIZE // _NBAND
_HS = 128
_NSLAB = _COLS_B // _HS
_CH1 = 320
_TB = 48
_PIECE = 2048


def _gather_body(kvt_hbm, loc_hbm, nope_hbm, tail_hbm,
                 locbuf, cmp, stage, slab_a, slab_b,
                 build, toks, oidx, tidx, gsem, wsem):
    wid = lax.axis_index("s") * _NC + lax.axis_index("c")
    band = wid >> 1
    half = wid & 1
    ch0 = half * 256
    col0 = band * _COLS_B
    lane = lax.iota(jnp.int32, 16)
    zero16 = jnp.zeros((16,), jnp.int32)

    off = jnp.int32(0)
    for p in range(N_TOK // _PIECE):
        pltpu.sync_copy(loc_hbm.at[pl.ds(p * _PIECE, _PIECE)], locbuf)

        def body0(v, o, _p=p):
            lv = locbuf[pl.ds(v * 16, 16)]
            m = (lv >> 12) == band
            tok = _p * _PIECE + v * 16 + lane
            pack = ((lv & (_COLS_B - 1)) << 14) | tok
            pref = plsc.cumsum(m.astype(jnp.int32))
            plsc.store_scatter(cmp, [o + pref - 1], pack, mask=m)
            return o + jnp.sum(m.astype(jnp.int32))

        off = lax.fori_loop(0, _PIECE // 16, body0, off)

    def flush():
        for g in range(_TB // 16):
            tv = toks[pl.ds(g * 16, 16)]
            oidx[0, pl.ds(g * 16, 16)] = tv * 4 + 2 * half
            oidx[1, pl.ds(g * 16, 16)] = tv * 4 + 2 * half + 1
            tidx[0, pl.ds(g * 16, 16)] = tv
        w0 = pltpu.async_copy(build.at[:, pl.ds(0, 128)],
                              nope_hbm.at[oidx.at[0]], wsem)
        w1 = pltpu.async_copy(build.at[:, pl.ds(128, 128)],
                              nope_hbm.at[oidx.at[1]], wsem)

        @pl.when(half == 1)
        def _():
            pltpu.async_copy(build.at[:, pl.ds(384, 128)],
                             tail_hbm.at[tidx.at[0]], wsem).wait()

        w0.wait()
        w1.wait()

    def process_slab(jj, slab, b):
        def body1(u, carry):
            b_in = carry
            e16 = cmp[pl.ds(u * 16, 16)]
            inb = (u * 16 + lane) < off
            m = ((e16 >> 14) >> 7 == jj) & inb
            pref = plsc.cumsum(m.astype(jnp.int32))
            plsc.store_scatter(stage, [pref - 1], e16, mask=m)
            cnt = jnp.sum(m.astype(jnp.int32))

            def body2(t, bb):
                e = stage[pl.ds(t, 16)][0]
                o = (e >> 14) & (_HS - 1)
                tok = e & (N_TOK - 1)
                plsc.store_scatter(toks, [bb + zero16], tok + zero16,
                                   mask=lane == 0)
                cidx = o + zero16
                bbv = bb + zero16
                for k in range(_CH1 // 16):
                    vals = plsc.load_gather(slab, [lane + k * 16, cidx])
                    plsc.store_scatter(build, [bbv, lane + k * 16], vals)
                    if k >= 12:
                        plsc.store_scatter(
                            build, [bbv, lane + (384 + (k - 12) * 16)], vals)
                bb = bb + 1

                @pl.when(bb == _TB)
                def _():
                    flush()

                return jnp.where(bb == _TB, 0, bb)

            return lax.fori_loop(0, cnt, body2, b_in)

        nvr = (off + 15) >> 4
        return lax.fori_loop(0, nvr, body1, b)

    def slab_copy(jj, buf):
        @pl.when(half == 0)
        def _():
            pltpu.async_copy(
                kvt_hbm.at[pl.ds(0, 256), pl.ds(pl.multiple_of(col0 + jj * _HS, 128), _HS)],
                buf.at[pl.ds(0, 256)], gsem)

        @pl.when(half == 1)
        def _():
            pltpu.async_copy(
                kvt_hbm.at[pl.ds(256, _CH1), pl.ds(pl.multiple_of(col0 + jj * _HS, 128), _HS)],
                buf.at[pl.ds(0, _CH1)], gsem)

    def slab_wait(buf):
        @pl.when(half == 0)
        def _():
            pltpu.make_async_copy(
                kvt_hbm.at[pl.ds(0, 256), pl.ds(0, _HS)],
                buf.at[pl.ds(0, 256)], gsem).wait()

        @pl.when(half == 1)
        def _():
            pltpu.make_async_copy(
                kvt_hbm.at[pl.ds(0, _CH1), pl.ds(0, _HS)],
                buf.at[pl.ds(0, _CH1)], gsem).wait()

    slab_copy(0, slab_a)

    def outer(i, b):
        jj_a = i * 2
        jj_b = i * 2 + 1
        slab_copy(jj_b, slab_b)
        slab_wait(slab_a)
        b = process_slab(jj_a, slab_a, b)
        slab_copy(jnp.minimum(jj_a + 2, _NSLAB - 1), slab_a)
        slab_wait(slab_b)
        return process_slab(jj_b, slab_b, b)

    b = lax.fori_loop(0, _NSLAB // 2, outer, jnp.int32(0))
    slab_wait(slab_a)

    @pl.when(b > 0)
    def _():
        def pad(s, _):
            t0 = toks[pl.ds(0, 16)][0]
            plsc.store_scatter(toks, [s + zero16], t0 + zero16,
                               mask=lane == 0)
            sv = s + zero16
            for m in range(512 // 16):
                v = plsc.load_gather(build, [zero16, lane + m * 16])
                plsc.store_scatter(build, [sv, lane + m * 16], v)
            return 0

        lax.fori_loop(b, _TB, pad, 0)
        flush()


@jax.jit
def _mla_gather(kvt, loc32):
    mesh = plsc.VectorSubcoreMesh(core_axis_name="c", subcore_axis_name="s")
    gather = functools.partial(
        pl.kernel,
        mesh=mesh,
        out_type=(
            jax.ShapeDtypeStruct((4 * N_TOK, 128), jnp.float32),
            jax.ShapeDtypeStruct((N_TOK, 128), jnp.float32),
        ),
        compiler_params=pltpu.CompilerParams(needs_layout_passes=False),
        scratch_types=[
            pltpu.VMEM((_PIECE,), jnp.int32),
            pltpu.VMEM((N_TOK + 16,), jnp.int32),
            pltpu.VMEM((32,), jnp.int32),
            pltpu.VMEM((_CH1, _HS), jnp.float32),
            pltpu.VMEM((_CH1, _HS), jnp.float32),
            pltpu.VMEM((_TB, 512), jnp.float32),
            pltpu.VMEM((_TB + 16,), jnp.int32),
            pltpu.VMEM((2, _TB), jnp.int32),
            pltpu.VMEM((1, _TB), jnp.int32),
            pltpu.SemaphoreType.DMA,
            pltpu.SemaphoreType.DMA,
        ],
    )(_gather_body)
    return gather(kvt, loc32)


def kernel(kv_buffer, loc, cache_k_nope, cache_k_rope):
    kv2d = kv_buffer.reshape(POOL_SIZE, ROW_DIM)
    kvt = kv2d.T
    loc32 = loc.astype(jnp.int32)
    nope4, tail = _mla_gather(kvt, loc32)
    nope = nope4.reshape(N_TOK, NOPE_DIM)
    rope = tail[:, 128 - ROPE_DIM:]
    return (nope.reshape(N_TOK, 1, NOPE_DIM).astype(cache_k_nope.dtype),
            rope.reshape(N_TOK, 1, ROPE_DIM).astype(cache_k_rope.dtype))


# speedup vs baseline: 1.9598x; 1.9598x over previous
import functools

import jax
import jax.numpy as jnp
from jax import lax
from jax.experimental import pallas as pl
from jax.experimental.pallas import tpu as pltpu
from jax.experimental.pallas import tpu_sc as plsc

POOL_SIZE = 65536
N_TOK = 16384
NOPE_DIM = 512
ROPE_DIM = 64
ROW_DIM = NOPE_DIM + ROPE_DIM

_NC, _NS = 2, 16
_NW = _NC * _NS
_B_PER_W = N_TOK // _NW
_CHUNK = 64
_N_CHUNKS = _B_PER_W // _CHUNK
_NBUF = 3


def _gather_body(kv_hbm, tail_hbm, loc_hbm, nope_hbm, rope_hbm,
                 idx_v, oidx_v, n0, n1, n2, t0, t1, t2, gsem, wsem):
    wid = lax.axis_index("s") * _NC + lax.axis_index("c")
    base = wid * _B_PER_W
    pltpu.sync_copy(loc_hbm.at[pl.ds(base, _B_PER_W)], idx_v)
    nbufs = (n0, n1, n2)
    tbufs = (t0, t1, t2)

    lane = lax.iota(jnp.int32, 16)
    for j in range(_N_CHUNKS):
        for k in range(4):
            for v in range(_CHUNK // 16):
                t0_ = base + j * _CHUNK + v * 16
                oidx_v[j, k, pl.ds(v * 16, 16)] = lane * 4 + (4 * t0_ + k)

    def start_gathers(j):
        idx_chunk = idx_v.at[pl.ds(j * _CHUNK, _CHUNK)]
        gn = pltpu.async_copy(kv_hbm.at[idx_chunk, pl.ds(0, NOPE_DIM)],
                              nbufs[j % _NBUF], gsem)
        gt = pltpu.async_copy(tail_hbm.at[idx_chunk], tbufs[j % _NBUF], gsem)
        return gn, gt

    def start_writes(j):
        nv = nbufs[j % _NBUF]
        tv = tbufs[j % _NBUF]
        ws = []
        for k in range(4):
            ws.append(pltpu.async_copy(nv.at[:, pl.ds(128 * k, 128)],
                                       nope_hbm.at[oidx_v.at[j, k]], wsem))
        row0 = base + j * _CHUNK
        ws.append(pltpu.async_copy(tv, rope_hbm.at[pl.ds(row0, _CHUNK)], wsem))
        return ws

    g = {0: start_gathers(0)}
    w = {}
    for j in range(_N_CHUNKS):
        if j + 1 < _N_CHUNKS:
            if j - 2 >= 0:
                for c in w.pop(j - 2):
                    c.wait()
            g[j + 1] = start_gathers(j + 1)
        for c in g.pop(j):
            c.wait()
        w[j] = start_writes(j)
    for j in sorted(w):
        for c in w.pop(j):
            c.wait()


@jax.jit
def _mla_gather(kv2d, kv_tail, loc32):
    mesh = plsc.VectorSubcoreMesh(core_axis_name="c", subcore_axis_name="s")
    gather = functools.partial(
        pl.kernel,
        mesh=mesh,
        out_type=(
            jax.ShapeDtypeStruct((4 * N_TOK, 128), jnp.float32),
            jax.ShapeDtypeStruct((N_TOK, 128), jnp.float32),
        ),
        scratch_types=[
            pltpu.VMEM((_B_PER_W,), jnp.int32),
            pltpu.VMEM((_N_CHUNKS, 4, _CHUNK), jnp.int32),
            pltpu.VMEM((_CHUNK, NOPE_DIM), jnp.float32),
            pltpu.VMEM((_CHUNK, NOPE_DIM), jnp.float32),
            pltpu.VMEM((_CHUNK, NOPE_DIM), jnp.float32),
            pltpu.VMEM((_CHUNK, 128), jnp.float32),
            pltpu.VMEM((_CHUNK, 128), jnp.float32),
            pltpu.VMEM((_CHUNK, 128), jnp.float32),
            pltpu.SemaphoreType.DMA,
            pltpu.SemaphoreType.DMA,
        ],
    )(_gather_body)
    return gather(kv2d, kv_tail, loc32)


def kernel(kv_buffer, loc, cache_k_nope, cache_k_rope):
    kv2d = kv_buffer.reshape(POOL_SIZE, ROW_DIM)
    kv_tail = kv2d[:, ROW_DIM - 128:]
    loc32 = loc.astype(jnp.int32)
    nope4, tail = _mla_gather(kv2d, kv_tail, loc32)
    nope = nope4.reshape(N_TOK, NOPE_DIM)
    rope = tail[:, 128 - ROPE_DIM:]
    return (nope.reshape(N_TOK, 1, NOPE_DIM).astype(cache_k_nope.dtype),
            rope.reshape(N_TOK, 1, ROPE_DIM).astype(cache_k_rope.dtype))
